# baseline (speedup 1.0000x reference)
import jax
import jax.numpy as jnp
from jax.experimental import pallas as pl
from jax.experimental.pallas import tpu as pltpu

N = 4096
NB = 8192
E = 32
H = 256
DEPTH = 10
BATCH = 64
NUM_ATOM_TYPES = 200
NUM_BOND_TYPES = 10

_ROWS = 512
_NBLK = N // _ROWS


def _split_fp8(hN):
    f32 = jnp.float32
    f8 = jnp.float8_e4m3fn
    t0 = hN.astype(f8)
    r0 = hN - t0.astype(f32)
    t1 = (r0 * 256.0).astype(f8)
    r1 = r0 - t1.astype(f32) * (1.0 / 256.0)
    t2 = (r1 * 65536.0).astype(f8)
    return jnp.concatenate([t0, t1, t2], axis=1)


def _mpn_body(adj_ref, af_ref, bi_ref, atom_embT_ref,
              msgW_ref, mbc_ref, Wih_ref, Whh_ref, bihc_ref, bhhc_ref,
              poolW_ref, pbc_ref,
              hT_out_ref, graphT_ref,
              mask_ref, scale_ref, flag_ref, bufT_ref, h2_ref, agg_ref,
              CT2_ref, bias_ref):
    l = pl.program_id(0)
    b = pl.program_id(1)
    f32 = jnp.float32
    bf16 = jnp.bfloat16
    f8 = jnp.float8_e4m3fn

    @pl.when(jnp.logical_and(l == 0, b == 0))
    def _init():
        af = af_ref[...]
        iota = jax.lax.broadcasted_iota(jnp.int32, (NUM_ATOM_TYPES, N), 0)
        ohT = (iota == af).astype(f32)
        h0T = jnp.dot(atom_embT_ref[...], ohT, preferred_element_type=f32)
        bufT_ref[0] = h0T
        h2_ref[0] = _split_fp8(jnp.transpose(h0T))

    @pl.when(jnp.logical_and(jnp.logical_and(l >= 1, l <= DEPTH), b == 1))
    def _setup():
        Wih = Wih_ref[0]
        W = msgW_ref[0]
        A_T = jnp.dot(Wih, W[:, :E], preferred_element_type=f32)
        B_T = jnp.dot(Wih, W[:, E:], preferred_element_type=f32)
        top = jnp.concatenate([A_T, B_T], axis=1)
        bot = jnp.concatenate([Whh_ref[0], jnp.zeros((3 * E, E), f32)],
                              axis=1)
        CT = jnp.concatenate([top, bot], axis=0)
        CThi = CT.astype(bf16)
        CTlo = (CT - CThi.astype(f32)).astype(bf16)
        CT2_ref[...] = jnp.concatenate([CThi, CTlo], axis=0)
        rowT = jnp.dot(Wih, mbc_ref[0], preferred_element_type=f32)
        bias_ref[...] = jnp.concatenate([rowT, bihc_ref[0], bhhc_ref[0]],
                                        axis=1)

    def _gate_chunk(L, c):
        cols = pl.ds(c * _ROWS, _ROWS)
        aggT = jnp.transpose(agg_ref[pl.ds(c * _ROWS, _ROWS), :])
        agg1 = (aggT[:E] + aggT[E:2 * E] * (1.0 / 256.0)
                + aggT[2 * E:] * (1.0 / 65536.0))
        aggs = agg1 * scale_ref[:, cols]
        hT = bufT_ref[L % 2, :, cols]
        XT = jnp.concatenate([hT, aggs], axis=0)
        XThi = XT.astype(bf16)
        XTlo = (XT - XThi.astype(f32)).astype(bf16)
        G1 = jnp.dot(CT2_ref[...], XThi, preferred_element_type=f32)
        G2 = jnp.dot(CT2_ref[:6 * E], XTlo, preferred_element_type=f32)
        GT = G1[:6 * E] + G1[6 * E:] + G2
        rowT = bias_ref[:, 0:1]
        bih = bias_ref[:, 1:2]
        bhh = bias_ref[:, 2:3]
        giT = flag_ref[:, cols] * (GT[:3 * E] + rowT) + bih
        ghT = GT[3 * E:] + bhh
        rT = jax.nn.sigmoid(giT[:E] + ghT[:E])
        zT = jax.nn.sigmoid(giT[E:2 * E] + ghT[E:2 * E])
        nT = jnp.tanh(giT[2 * E:] + rT * ghT[2 * E:])
        hnT = (1.0 - zT) * nT + zT * hT
        bufT_ref[(L + 1) % 2, :, cols] = hnT
        h2_ref[(L + 1) % 2, pl.ds(c * _ROWS, _ROWS), :] = _split_fp8(
            jnp.transpose(hnT))

        @pl.when(L == DEPTH - 1)
        def _emit():
            hT_out_ref[:, cols] = hnT

    @pl.when(jnp.logical_and(l >= 2, b == 0))
    def _chunk7():
        _gate_chunk(l - 2, _NBLK - 1)

    @pl.when(l == 1)
    def _convert():
        a = adj_ref[...]
        m = a != 0
        mask_ref[pl.ds(b * _ROWS, _ROWS), :] = m.astype(f8)
        deg = jnp.sum(m.astype(f32), axis=1, keepdims=True)
        has = deg > 0.0
        safe = jnp.where(has, deg, 1.0)
        sc = jnp.where(has, 1.0 / safe, 0.0)
        scale_ref[:, pl.ds(b * _ROWS, _ROWS)] = jnp.transpose(sc)
        flag_ref[:, pl.ds(b * _ROWS, _ROWS)] = jnp.transpose(has.astype(f32))

    @pl.when(jnp.logical_and(l >= 1, l <= DEPTH))
    def _layer():
        mblk = mask_ref[pl.ds(b * _ROWS, _ROWS), :]
        acc = jnp.dot(mblk, h2_ref[(l - 1) % 2],
                      preferred_element_type=f32)
        agg_ref[pl.ds(b * _ROWS, _ROWS), :] = acc

    @pl.when(jnp.logical_and(jnp.logical_and(l >= 1, l <= DEPTH), b >= 1))
    def _chunk():
        _gate_chunk(l - 1, b - 1)

    @pl.when(jnp.logical_and(l == DEPTH + 1, b == 0))
    def _pool():
        hfT = bufT_ref[DEPTH % 2]
        bi = bi_ref[...]
        iota = jax.lax.broadcasted_iota(jnp.int32, (N, BATCH), 1)
        PT = (iota == bi).astype(f32)
        countsT = jnp.sum(PT, axis=0, keepdims=True)
        sumsT = jnp.dot(hfT, PT, preferred_element_type=f32)
        inv = jnp.where(countsT > 0.0,
                        1.0 / jnp.where(countsT > 0.0, countsT, 1.0), 0.0)
        meansT = sumsT * inv
        graphT_ref[...] = (jnp.dot(poolW_ref[...], meansT,
                                   preferred_element_type=f32) + pbc_ref[...])


def _bond_body(bf_ref, bond_emb_ref, out_ref):
    bfi = bf_ref[...]
    iota = jax.lax.broadcasted_iota(jnp.int32, (NB, NUM_BOND_TYPES), 1)
    oh = (bfi == iota).astype(jnp.float32)
    out_ref[...] = jnp.dot(oh, bond_emb_ref[...],
                           preferred_element_type=jnp.float32)


def kernel(atom_features, bond_features, adjacency_matrix, batch_indices,
           atom_emb, bond_emb, msg_W, msg_b,
           gru_Wih, gru_Whh, gru_bih, gru_bhh, pool_W, pool_b):
    af = atom_features.reshape(1, N).astype(jnp.int32)
    bf = bond_features.reshape(NB, 1).astype(jnp.int32)
    bi = batch_indices.reshape(N, 1).astype(jnp.int32)
    atom_embT = atom_emb.T
    mbc = msg_b.reshape(DEPTH, H, 1)
    bihc = gru_bih.reshape(DEPTH, 3 * E, 1)
    bhhc = gru_bhh.reshape(DEPTH, 3 * E, 1)
    pbc = pool_b.reshape(H, 1)

    def _wmap(nd):
        def im(l, b):
            lw = jnp.clip(l - 1, 0, DEPTH - 1)
            return (lw,) + (0,) * (nd - 1)
        return im

    def _adj_map(l, b):
        return (jnp.where(l >= 2, _NBLK - 1, jnp.where(l == 1, b, 0)), 0)

    hT_out, graphT = pl.pallas_call(
        _mpn_body,
        grid=(DEPTH + 2, _NBLK),
        in_specs=[
            pl.BlockSpec((_ROWS, N), _adj_map),
            pl.BlockSpec((1, N), lambda l, b: (0, 0)),
            pl.BlockSpec((N, 1), lambda l, b: (0, 0)),
            pl.BlockSpec((E, NUM_ATOM_TYPES), lambda l, b: (0, 0)),
            pl.BlockSpec((1, H, 2 * E), _wmap(3)),
            pl.BlockSpec((1, H, 1), _wmap(3)),
            pl.BlockSpec((1, 3 * E, H), _wmap(3)),
            pl.BlockSpec((1, 3 * E, E), _wmap(3)),
            pl.BlockSpec((1, 3 * E, 1), _wmap(3)),
            pl.BlockSpec((1, 3 * E, 1), _wmap(3)),
            pl.BlockSpec((H, E), lambda l, b: (0, 0)),
            pl.BlockSpec((H, 1), lambda l, b: (0, 0)),
        ],
        out_specs=[
            pl.BlockSpec((E, N), lambda l, b: (0, 0)),
            pl.BlockSpec((H, BATCH), lambda l, b: (0, 0)),
        ],
        out_shape=[
            jax.ShapeDtypeStruct((E, N), jnp.float32),
            jax.ShapeDtypeStruct((H, BATCH), jnp.float32),
        ],
        scratch_shapes=[
            pltpu.VMEM((N, N), jnp.float8_e4m3fn),
            pltpu.VMEM((1, N), jnp.float32),
            pltpu.VMEM((1, N), jnp.float32),
            pltpu.VMEM((2, E, N), jnp.float32),
            pltpu.VMEM((2, N, 3 * E), jnp.float8_e4m3fn),
            pltpu.VMEM((N, 3 * E), jnp.float32),
            pltpu.VMEM((12 * E, 2 * E), jnp.bfloat16),
            pltpu.VMEM((3 * E, 3), jnp.float32),
        ],
    )(adjacency_matrix, af, bi, atom_embT, msg_W, mbc, gru_Wih, gru_Whh,
      bihc, bhhc, pool_W, pbc)

    bond_out = pl.pallas_call(
        _bond_body,
        out_shape=jax.ShapeDtypeStruct((NB, E), jnp.float32),
    )(bf, bond_emb)

    return (hT_out.T, bond_out, graphT.T)
